# baseline (device time: 40459 ns/iter reference)
import functools

import jax
import jax.numpy as jnp
from jax import lax
from jax.experimental import pallas as pl
from jax.experimental.pallas import tpu as pltpu

N_DEV = 16
PLANE = 4


def kernel(x, Win0, Wout0, Win1, Wout1, Win2, Wout2):
    b_per, d = x.shape
    hid = Win0.shape[1]

    def body(x_ref, win0_ref, wout0_ref, win1_ref, wout1_ref, win2_ref,
             wout2_ref, out_ref, xch_ref, prec_ref, pch_ref,
             winq0, woutq0, winq1, woutq1, winq2, woutq2,
             wsend_sems, wrecv_sems, agsend_sems, psend_sems,
             agrecv_sems, prrecv_sems):
        my = lax.axis_index("i")
        P = my // PLANE
        Q = my % PLANE

        def plane_peer(o):
            return PLANE * P + (Q + o) % PLANE

        def col_peer(o):
            return PLANE * ((P + o) % PLANE) + Q

        barrier_sem = pltpu.get_barrier_semaphore()
        for o in range(1, PLANE):
            for peer in (plane_peer(o), col_peer(o)):
                pl.semaphore_signal(
                    barrier_sem, inc=1, device_id=(peer,),
                    device_id_type=pl.DeviceIdType.MESH)
        pl.semaphore_wait(barrier_sem, 2 * (PLANE - 1))

        weight_sends = []
        ag_pending = []
        p_pending = {o: None for o in range(1, PLANE)}

        def wait_prev_bcast():
            for r in ag_pending:
                r.wait_send()
            ag_pending.clear()

        def bcast_my_chunk():
            for o in range(1, PLANE):
                rdma = pltpu.make_async_remote_copy(
                    src_ref=xch_ref.at[pl.ds(Q, 1)],
                    dst_ref=xch_ref.at[pl.ds(Q, 1)],
                    send_sem=agsend_sems.at[o],
                    recv_sem=agrecv_sems.at[Q],
                    device_id=(plane_peer(o),),
                    device_id_type=pl.DeviceIdType.MESH)
                rdma.start()
                ag_pending.append(rdma)

        def wait_chunk(c, o):
            rdma = pltpu.make_async_remote_copy(
                src_ref=xch_ref.at[pl.ds(c, 1)],
                dst_ref=xch_ref.at[pl.ds(c, 1)],
                send_sem=agsend_sems.at[o],
                recv_sem=agrecv_sems.at[c],
                device_id=(my,),
                device_id_type=pl.DeviceIdType.MESH)
            rdma.wait_recv()

        def wait_prev_partial_send(o):
            if p_pending[o] is not None:
                p_pending[o].wait_send()
                p_pending[o] = None

        def send_partial(c, o):
            rdma = pltpu.make_async_remote_copy(
                src_ref=pch_ref.at[pl.ds(c, 1)],
                dst_ref=prec_ref.at[pl.ds(Q, 1)],
                send_sem=psend_sems.at[o],
                recv_sem=prrecv_sems.at[Q],
                device_id=(plane_peer(o),),
                device_id_type=pl.DeviceIdType.MESH)
            rdma.start()
            p_pending[o] = rdma

        def wait_partials():
            for o in range(1, PLANE):
                s = (Q + o) % PLANE
                rdma = pltpu.make_async_remote_copy(
                    src_ref=prec_ref.at[pl.ds(s, 1)],
                    dst_ref=prec_ref.at[pl.ds(s, 1)],
                    send_sem=psend_sems.at[o],
                    recv_sem=prrecv_sems.at[s],
                    device_id=(my,),
                    device_id_type=pl.DeviceIdType.MESH)
                rdma.wait_recv()

        xch_ref[pl.ds(Q, 1)] = x_ref[...].astype(jnp.bfloat16)[None]
        bcast_my_chunk()

        wq = [(winq0, woutq0), (winq1, woutq1), (winq2, woutq2)]
        win_in = [(win0_ref, wout0_ref), (win1_ref, wout1_ref),
                  (win2_ref, wout2_ref)]
        for li in range(3):
            for wi in range(2):
                buf = wq[li][wi]
                buf[pl.ds(P, 1)] = win_in[li][wi][...].astype(jnp.bfloat16)[None]
                w = 2 * li + wi
                for o in range(1, PLANE):
                    rdma = pltpu.make_async_remote_copy(
                        src_ref=buf.at[pl.ds(P, 1)],
                        dst_ref=buf.at[pl.ds(P, 1)],
                        send_sem=wsend_sems.at[w, o],
                        recv_sem=wrecv_sems.at[w, P],
                        device_id=(col_peer(o),),
                        device_id_type=pl.DeviceIdType.MESH)
                    rdma.start()
                    weight_sends.append(rdma)

        def wait_weight_slot(li, s, o):
            for wi in range(2):
                buf = wq[li][wi]
                w = 2 * li + wi
                rdma = pltpu.make_async_remote_copy(
                    src_ref=buf.at[pl.ds(s, 1)],
                    dst_ref=buf.at[pl.ds(s, 1)],
                    send_sem=wsend_sems.at[w, o],
                    recv_sem=wrecv_sems.at[w, s],
                    device_id=(my,),
                    device_id_type=pl.DeviceIdType.MESH)
                rdma.wait_recv()

        def quarter_mlp(li, xc):
            winq, woutq = wq[li]
            acc = None
            for p in range(PLANE):
                h = jnp.dot(xc, winq[p], preferred_element_type=jnp.float32)
                h = jnp.maximum(h, 0.0).astype(jnp.bfloat16)
                pp = jnp.dot(h, woutq[p], preferred_element_type=jnp.float32)
                acc = pp if acc is None else acc + pp
            return acc

        def quarter_mlp_l0(xc):
            winq, woutq = wq[0]
            acc = None
            for o in range(PLANE):
                s = (P + o) % PLANE
                if o > 0:
                    wait_weight_slot(0, s, o)
                h = jnp.dot(xc, winq[pl.ds(s, 1)].reshape(d, hid),
                            preferred_element_type=jnp.float32)
                h = jnp.maximum(h, 0.0).astype(jnp.bfloat16)
                pp = jnp.dot(h, woutq[pl.ds(s, 1)].reshape(hid, d),
                             preferred_element_type=jnp.float32)
                acc = pp if acc is None else acc + pp
            return acc

        prec_ref[pl.ds(Q, 1)] = quarter_mlp_l0(
            xch_ref[pl.ds(Q, 1)].reshape(b_per, d)).astype(jnp.bfloat16)[None]
        for o in range(1, PLANE):
            c = (Q + o) % PLANE
            wait_chunk(c, o)
            pch_ref[pl.ds(c, 1)] = quarter_mlp(
                0, xch_ref[pl.ds(c, 1)].reshape(b_per, d)).astype(
                    jnp.bfloat16)[None]
            send_partial(c, o)

        for li in (1, 2):
            wait_partials()
            red = jnp.sum(prec_ref[...].astype(jnp.float32), axis=0)
            wait_prev_bcast()
            xch_ref[pl.ds(Q, 1)] = red.astype(jnp.bfloat16)[None]
            bcast_my_chunk()
            for o in range(1, PLANE):
                wait_weight_slot(li, (P + o) % PLANE, o)
            prec_ref[pl.ds(Q, 1)] = quarter_mlp(
                li, xch_ref[pl.ds(Q, 1)].reshape(b_per, d)).astype(
                    jnp.bfloat16)[None]
            for o in range(1, PLANE):
                c = (Q + o) % PLANE
                wait_chunk(c, o)
                wait_prev_partial_send(o)
                pch_ref[pl.ds(c, 1)] = quarter_mlp(
                    li, xch_ref[pl.ds(c, 1)].reshape(b_per, d)).astype(
                        jnp.bfloat16)[None]
                send_partial(c, o)

        wait_partials()
        out_ref[...] = jnp.sum(prec_ref[...].astype(jnp.float32), axis=0)

        for r in ag_pending:
            r.wait_send()
        for o in range(1, PLANE):
            if p_pending[o] is not None:
                p_pending[o].wait_send()
        for r in weight_sends:
            r.wait_send()

        @functools.partial(pl.run_scoped,
                           second_barrier=pltpu.SemaphoreType.REGULAR)
        def _(second_barrier):
            for o in range(1, PLANE):
                pl.semaphore_signal(
                    second_barrier, inc=1, device_id=(col_peer(o),),
                    device_id_type=pl.DeviceIdType.MESH)
            pl.semaphore_wait(second_barrier, PLANE - 1)

    return pl.pallas_call(
        body,
        out_shape=jax.ShapeDtypeStruct((b_per, d), jnp.float32),
        in_specs=[pl.BlockSpec(memory_space=pltpu.VMEM)] * 7,
        out_specs=pl.BlockSpec(memory_space=pltpu.VMEM),
        scratch_shapes=[
            pltpu.VMEM((PLANE, b_per, d), jnp.bfloat16),
            pltpu.VMEM((PLANE, b_per, d), jnp.bfloat16),
            pltpu.VMEM((PLANE, b_per, d), jnp.bfloat16),
            pltpu.VMEM((PLANE, d, hid), jnp.bfloat16),
            pltpu.VMEM((PLANE, hid, d), jnp.bfloat16),
            pltpu.VMEM((PLANE, d, hid), jnp.bfloat16),
            pltpu.VMEM((PLANE, hid, d), jnp.bfloat16),
            pltpu.VMEM((PLANE, d, hid), jnp.bfloat16),
            pltpu.VMEM((PLANE, hid, d), jnp.bfloat16),
            pltpu.SemaphoreType.DMA((6, PLANE)),
            pltpu.SemaphoreType.DMA((6, PLANE)),
            pltpu.SemaphoreType.DMA((PLANE,)),
            pltpu.SemaphoreType.DMA((PLANE,)),
            pltpu.SemaphoreType.DMA((PLANE,)),
            pltpu.SemaphoreType.DMA((PLANE,)),
        ],
        compiler_params=pltpu.CompilerParams(collective_id=0),
    )(x, Win0, Wout0, Win1, Wout1, Win2, Wout2)


# device time: 40077 ns/iter; 1.0095x vs baseline; 1.0095x over previous
import functools

import jax
import jax.numpy as jnp
from jax import lax
from jax.experimental import pallas as pl
from jax.experimental.pallas import tpu as pltpu

N_DEV = 16
PLANE = 4


def kernel(x, Win0, Wout0, Win1, Wout1, Win2, Wout2):
    b_per, d = x.shape
    hid = Win0.shape[1]
    wflat = d * hid

    def body(x_ref, win0_ref, wout0_ref, win1_ref, wout1_ref, win2_ref,
             wout2_ref, out_ref, xblock_ref, precv0_ref, precv1_ref,
             psend_ref, rsb_ref, winq0, woutq0, winq1, woutq1, winq2,
             woutq2, wsend_sems, wrecv_sems, agsend_sems, psend_sems,
             agrecv_sems, prrecv_sems, rssend_sems, rsrecv_sems):
        my = lax.axis_index("i")
        P = my // PLANE
        Q = my % PLANE

        def plane_peer(o):
            return PLANE * P + (Q + o) % PLANE

        def col_peer(o):
            return PLANE * ((P + o) % PLANE) + Q

        barrier_sem = pltpu.get_barrier_semaphore()
        for o in range(1, PLANE):
            for peer in (plane_peer(o), col_peer(o)):
                pl.semaphore_signal(
                    barrier_sem, inc=1, device_id=(peer,),
                    device_id_type=pl.DeviceIdType.MESH)
        pl.semaphore_wait(barrier_sem, 2 * (PLANE - 1))

        xblock_ref[pl.ds(Q, 1)] = x_ref[...].astype(jnp.bfloat16)[None]
        xag_sends = []
        for o in range(1, PLANE):
            rdma = pltpu.make_async_remote_copy(
                src_ref=xblock_ref.at[pl.ds(Q, 1)],
                dst_ref=xblock_ref.at[pl.ds(Q, 1)],
                send_sem=agsend_sems.at[o],
                recv_sem=agrecv_sems.at[Q],
                device_id=(plane_peer(o),),
                device_id_type=pl.DeviceIdType.MESH)
            rdma.start()
            xag_sends.append(rdma)

        wq = [(winq0, woutq0), (winq1, woutq1), (winq2, woutq2)]
        win_in = [(win0_ref, wout0_ref), (win1_ref, wout1_ref),
                  (win2_ref, wout2_ref)]
        w_sends = []
        for li in range(3):
            for wi in range(2):
                buf = wq[li][wi]
                buf[pl.ds(P, 1)] = win_in[li][wi][...].astype(
                    jnp.bfloat16)[None]
                w = 2 * li + wi
                for o in range(1, PLANE):
                    rdma = pltpu.make_async_remote_copy(
                        src_ref=buf.at[pl.ds(P, 1)],
                        dst_ref=buf.at[pl.ds(P, 1)],
                        send_sem=wsend_sems.at[w, o],
                        recv_sem=wrecv_sems.at[w, P],
                        device_id=(col_peer(o),),
                        device_id_type=pl.DeviceIdType.MESH)
                    rdma.start()
                    w_sends.append(rdma)

        def layer(li):
            winq, woutq = wq[li]
            xb = xblock_ref[...].reshape(PLANE * b_per, d)
            acc = None
            for o in range(PLANE):
                s = (P + o) % PLANE
                if o > 0:
                    for wi, buf in ((0, winq), (1, woutq)):
                        rdma = pltpu.make_async_remote_copy(
                            src_ref=buf.at[pl.ds(s, 1)],
                            dst_ref=buf.at[pl.ds(s, 1)],
                            send_sem=wsend_sems.at[2 * li + wi, o],
                            recv_sem=wrecv_sems.at[2 * li + wi, s],
                            device_id=(my,),
                            device_id_type=pl.DeviceIdType.MESH)
                        rdma.wait_recv()
                win = winq[pl.ds(s, 1)].reshape(d, hid)
                wout = woutq[pl.ds(s, 1)].reshape(hid, d)
                h = jnp.dot(xb, win, preferred_element_type=jnp.float32)
                h = jnp.maximum(h, 0.0).astype(jnp.bfloat16)
                pp = jnp.dot(h, wout, preferred_element_type=jnp.float32)
                acc = pp if acc is None else acc + pp
            return acc

        pexch_sends = {0: [], 1: []}
        for li in range(2):
            if li == 0:
                for o in range(1, PLANE):
                    s = (Q + o) % PLANE
                    rdma = pltpu.make_async_remote_copy(
                        src_ref=xblock_ref.at[pl.ds(s, 1)],
                        dst_ref=xblock_ref.at[pl.ds(s, 1)],
                        send_sem=agsend_sems.at[o],
                        recv_sem=agrecv_sems.at[s],
                        device_id=(my,),
                        device_id_type=pl.DeviceIdType.MESH)
                    rdma.wait_recv()
            partial = layer(li)
            precv = (precv0_ref, precv1_ref)[li]
            if li == 1:
                for r in pexch_sends[0]:
                    r.wait_send()
            psend_ref[...] = partial.reshape(1, PLANE, b_per, d).astype(
                jnp.bfloat16)
            for o in range(1, PLANE):
                rdma = pltpu.make_async_remote_copy(
                    src_ref=psend_ref.at[pl.ds(0, 1)],
                    dst_ref=precv.at[pl.ds(Q, 1)],
                    send_sem=psend_sems.at[li, o],
                    recv_sem=prrecv_sems.at[li, Q],
                    device_id=(plane_peer(o),),
                    device_id_type=pl.DeviceIdType.MESH)
                rdma.start()
                pexch_sends[li].append(rdma)
            precv[pl.ds(Q, 1)] = psend_ref[...]
            for o in range(1, PLANE):
                s = (Q + o) % PLANE
                rdma = pltpu.make_async_remote_copy(
                    src_ref=precv.at[pl.ds(s, 1)],
                    dst_ref=precv.at[pl.ds(s, 1)],
                    send_sem=psend_sems.at[li, o],
                    recv_sem=prrecv_sems.at[li, s],
                    device_id=(my,),
                    device_id_type=pl.DeviceIdType.MESH)
                rdma.wait_recv()
            red = jnp.sum(precv[...].astype(jnp.float32), axis=0)
            if li == 0:
                for r in xag_sends:
                    r.wait_send()
            xblock_ref[...] = red.astype(jnp.bfloat16)

        partial = layer(2)
        for r in pexch_sends[1]:
            r.wait_send()
        psend_ref[...] = partial.reshape(1, PLANE, b_per, d).astype(
            jnp.bfloat16)
        rsb_ref[pl.ds(Q, 1)] = psend_ref[pl.ds(0, 1), pl.ds(Q, 1)]
        rs_sends = []
        for o in range(1, PLANE):
            s = (Q + o) % PLANE
            rdma = pltpu.make_async_remote_copy(
                src_ref=psend_ref.at[pl.ds(0, 1), pl.ds(s, 1)],
                dst_ref=rsb_ref.at[pl.ds(Q, 1)],
                send_sem=rssend_sems.at[o],
                recv_sem=rsrecv_sems.at[Q],
                device_id=(plane_peer(o),),
                device_id_type=pl.DeviceIdType.MESH)
            rdma.start()
            rs_sends.append(rdma)
        for o in range(1, PLANE):
            s = (Q + o) % PLANE
            rdma = pltpu.make_async_remote_copy(
                src_ref=rsb_ref.at[pl.ds(s, 1)],
                dst_ref=rsb_ref.at[pl.ds(s, 1)],
                send_sem=rssend_sems.at[o],
                recv_sem=rsrecv_sems.at[s],
                device_id=(my,),
                device_id_type=pl.DeviceIdType.MESH)
            rdma.wait_recv()
        out_ref[...] = jnp.sum(rsb_ref[...].astype(jnp.float32),
                               axis=(0, 1))

        for r in w_sends + rs_sends:
            r.wait_send()

        @functools.partial(pl.run_scoped,
                           second_barrier=pltpu.SemaphoreType.REGULAR)
        def _(second_barrier):
            for o in range(1, PLANE):
                pl.semaphore_signal(
                    second_barrier, inc=1, device_id=(col_peer(o),),
                    device_id_type=pl.DeviceIdType.MESH)
            pl.semaphore_wait(second_barrier, PLANE - 1)

    return pl.pallas_call(
        body,
        out_shape=jax.ShapeDtypeStruct((b_per, d), jnp.float32),
        in_specs=[pl.BlockSpec(memory_space=pltpu.VMEM)] * 7,
        out_specs=pl.BlockSpec(memory_space=pltpu.VMEM),
        scratch_shapes=[
            pltpu.VMEM((PLANE, b_per, d), jnp.bfloat16),
            pltpu.VMEM((PLANE, PLANE, b_per, d), jnp.bfloat16),
            pltpu.VMEM((PLANE, PLANE, b_per, d), jnp.bfloat16),
            pltpu.VMEM((1, PLANE, b_per, d), jnp.bfloat16),
            pltpu.VMEM((PLANE, 1, b_per, d), jnp.bfloat16),
            pltpu.VMEM((PLANE, d, hid), jnp.bfloat16),
            pltpu.VMEM((PLANE, hid, d), jnp.bfloat16),
            pltpu.VMEM((PLANE, d, hid), jnp.bfloat16),
            pltpu.VMEM((PLANE, hid, d), jnp.bfloat16),
            pltpu.VMEM((PLANE, d, hid), jnp.bfloat16),
            pltpu.VMEM((PLANE, hid, d), jnp.bfloat16),
            pltpu.SemaphoreType.DMA((6, PLANE)),
            pltpu.SemaphoreType.DMA((6, PLANE)),
            pltpu.SemaphoreType.DMA((PLANE,)),
            pltpu.SemaphoreType.DMA((2, PLANE)),
            pltpu.SemaphoreType.DMA((PLANE,)),
            pltpu.SemaphoreType.DMA((2, PLANE)),
            pltpu.SemaphoreType.DMA((PLANE,)),
            pltpu.SemaphoreType.DMA((PLANE,)),
        ],
        compiler_params=pltpu.CompilerParams(collective_id=0),
    )(x, Win0, Wout0, Win1, Wout1, Win2, Wout2)
